# baseline (speedup 1.0000x reference)
import functools

import jax
import jax.numpy as jnp
from jax import lax
from jax.experimental import pallas as pl
from jax.experimental.pallas import tpu as pltpu
from jax.experimental.pallas import tpu_sc as plsc

_NC, _NS, _LANES = 2, 16, 16
_NW = _NC * _NS

_D = 64


def _make_sc_gather(n_tokens, chunk=128, nbuf=5):
    npw = n_tokens // _NW
    nchunk = npw // chunk
    ngroup = nchunk // nbuf
    mesh = plsc.VectorSubcoreMesh(core_axis_name="c", subcore_axis_name="s")

    panel = 3200
    scratch = (
        [pltpu.VMEM((chunk,), jnp.int32) for _ in range(nbuf)]
        + [pltpu.VMEM((chunk, _D), jnp.float32) for _ in range(nbuf)]
        + [pltpu.SemaphoreType.DMA for _ in range(2 * nbuf)]
    )

    @functools.partial(
        pl.kernel,
        out_type=jax.ShapeDtypeStruct((n_tokens // 2, 2 * _D), jnp.float32),
        mesh=mesh,
        scratch_types=scratch,
        compiler_params=pltpu.CompilerParams(
            needs_layout_passes=False, use_tc_tiling_on_sc=False
        ),
    )
    def sc_gather(idx_hbm, table_hbm, out_hbm, *scr):
        idx_v = scr[0:nbuf]
        rows_v = scr[nbuf:2 * nbuf]
        gsem = scr[2 * nbuf:3 * nbuf]
        osem = scr[3 * nbuf:4 * nbuf]

        wid = lax.axis_index("s") * _NC + lax.axis_index("c")
        base = wid * npw

        def stage(s, slot):
            pltpu.sync_copy(idx_hbm.at[pl.ds(base + s * chunk, chunk)],
                            idx_v[slot])
            pltpu.async_copy(table_hbm.at[idx_v[slot]], rows_v[slot],
                             gsem[slot])

        def out_copy(s, slot, sem):
            t0 = base + s * chunk
            rem = t0 % (2 * panel)
            odd = rem >= panel
            r0 = (t0 // (2 * panel)) * panel + rem - jnp.where(odd, panel, 0)
            lane0 = jnp.where(odd, _D, 0)
            return pltpu.make_async_copy(
                rows_v[slot],
                out_hbm.at[pl.ds(r0, chunk), pl.ds(lane0, _D)],
                sem)

        ahead = 2
        for b in range(ahead):
            stage(b, b)

        def group_body(g, _):
            for b in range(nbuf):
                s = g * nbuf + b
                slot_n = (b + ahead) % nbuf

                @pl.when(s + ahead < nchunk)
                def _():
                    pltpu.sync_copy(
                        idx_hbm.at[pl.ds(base + (s + ahead) * chunk, chunk)],
                        idx_v[slot_n])

                    @pl.when(s + ahead - nbuf >= 0)
                    def _():
                        out_copy(0, slot_n, osem[slot_n]).wait()

                    pltpu.async_copy(table_hbm.at[idx_v[slot_n]],
                                     rows_v[slot_n], gsem[slot_n])

                pltpu.make_async_copy(
                    table_hbm.at[idx_v[b]], rows_v[b], gsem[b]).wait()
                out_copy(s, b, osem[b]).start()
            return 0

        lax.fori_loop(0, ngroup, group_body, 0)

        for b in range(nbuf):
            out_copy(0, b, osem[b]).wait()

    return sc_gather


def _xs_body(ids_ref, out_ref):
    ids = ids_ref[...]
    seq = ids.shape[1]
    tl = ((ids % 8) + 1).astype(jnp.float32)
    r = lax.broadcasted_iota(jnp.int32, (seq, seq), 0)
    c = lax.broadcasted_iota(jnp.int32, (seq, seq), 1)
    tri = (r < c).astype(jnp.float32)
    out_ref[...] = jnp.dot(
        tl, tri,
        preferred_element_type=jnp.float32,
        precision=lax.Precision.HIGHEST,
    ) * (1.0 / seq)


def _xs_compute(ids):
    b, seq = ids.shape
    blk_b = 512
    grid = b // blk_b
    return pl.pallas_call(
        _xs_body,
        grid=(grid,),
        in_specs=[pl.BlockSpec((blk_b, seq), lambda i: (i, 0))],
        out_specs=pl.BlockSpec((blk_b, seq), lambda i: (i, 0)),
        out_shape=jax.ShapeDtypeStruct((b, seq), jnp.float32),
    )(ids)


def kernel(input_ids, token_table, pos_W, pos_b):
    b, seq = input_ids.shape
    ids = input_ids.astype(jnp.int32)
    idsf = ids.reshape(b * seq)
    g2 = _make_sc_gather(b * seq)(idsf, token_table)
    xs = _xs_compute(ids)

    n = b * seq
    g3 = g2.reshape(n // 6400, 3200, 2 * _D)
    toks = jnp.stack([g3[:, :, :_D], g3[:, :, _D:]], axis=1)
    toks = toks.reshape(b, seq, _D)

    u = pos_W[0] + pos_W[1]
    v = pos_W[2] + pos_W[3]
    ws = ((ids % 8) + 1).astype(jnp.float32) * (1.0 / seq)
    return toks + xs[..., None] * u + ws[..., None] * v + pos_b


# speedup vs baseline: 2.3652x; 2.3652x over previous
import functools

import jax
import jax.numpy as jnp
from jax import lax
from jax.experimental import pallas as pl
from jax.experimental.pallas import tpu as pltpu
from jax.experimental.pallas import tpu_sc as plsc

_NC, _NS, _LANES = 2, 16, 16
_NW = _NC * _NS

_D = 64


def _make_sc_gather(n_tokens, chunk=128, nbuf=5):
    npw = n_tokens // _NW
    nchunk = npw // chunk
    ngroup = nchunk // nbuf
    mesh = plsc.VectorSubcoreMesh(core_axis_name="c", subcore_axis_name="s")

    panel = 3200
    scratch = (
        [pltpu.VMEM((chunk,), jnp.int32) for _ in range(nbuf)]
        + [pltpu.VMEM((chunk, _D), jnp.float32) for _ in range(nbuf)]
        + [pltpu.SemaphoreType.DMA for _ in range(2 * nbuf)]
    )

    @functools.partial(
        pl.kernel,
        out_type=jax.ShapeDtypeStruct((n_tokens // 2, 2 * _D), jnp.float32),
        mesh=mesh,
        scratch_types=scratch,
        compiler_params=pltpu.CompilerParams(
            needs_layout_passes=False, use_tc_tiling_on_sc=False
        ),
    )
    def sc_gather(idx_hbm, table_hbm, out_hbm, *scr):
        idx_v = scr[0:nbuf]
        rows_v = scr[nbuf:2 * nbuf]
        gsem = scr[2 * nbuf:3 * nbuf]
        osem = scr[3 * nbuf:4 * nbuf]

        wid = lax.axis_index("s") * _NC + lax.axis_index("c")
        base = wid * npw

        def stage(s, slot):
            pltpu.sync_copy(idx_hbm.at[pl.ds(base + s * chunk, chunk)],
                            idx_v[slot])
            pltpu.async_copy(table_hbm.at[idx_v[slot]], rows_v[slot],
                             gsem[slot])

        def out_copy(s, slot, sem):
            t0 = base + s * chunk
            rem = t0 % (2 * panel)
            odd = rem >= panel
            r0 = (t0 // (2 * panel)) * panel + rem - jnp.where(odd, panel, 0)
            lane0 = jnp.where(odd, _D, 0)
            return pltpu.make_async_copy(
                rows_v[slot],
                out_hbm.at[pl.ds(r0, chunk), pl.ds(lane0, _D)],
                sem)

        ahead = 2
        for b in range(ahead):
            stage(b, b)

        def group_body(g, _):
            for b in range(nbuf):
                s = g * nbuf + b
                slot_n = (b + ahead) % nbuf

                @pl.when(s + ahead < nchunk)
                def _():
                    pltpu.sync_copy(
                        idx_hbm.at[pl.ds(base + (s + ahead) * chunk, chunk)],
                        idx_v[slot_n])

                    @pl.when(s + ahead - nbuf >= 0)
                    def _():
                        out_copy(0, slot_n, osem[slot_n]).wait()

                    pltpu.async_copy(table_hbm.at[idx_v[slot_n]],
                                     rows_v[slot_n], gsem[slot_n])

                pltpu.make_async_copy(
                    table_hbm.at[idx_v[b]], rows_v[b], gsem[b]).wait()
                out_copy(s, b, osem[b]).start()
            return 0

        lax.fori_loop(0, ngroup, group_body, 0)

        for b in range(nbuf):
            out_copy(0, b, osem[b]).wait()

    return sc_gather


def _finalize_body(ids_ref, g2_ref, pw_ref, pb_ref, out_ref):
    bb, seq = ids_ref.shape
    hh = bb // 2
    x2 = g2_ref[...]

    ids = ids_ref[...]
    tl = ((ids % 8) + 1).astype(jnp.float32)
    r = lax.broadcasted_iota(jnp.int32, (seq, seq), 0)
    c = lax.broadcasted_iota(jnp.int32, (seq, seq), 1)
    tri = (r < c).astype(jnp.float32)
    start = jnp.dot(tl, tri, preferred_element_type=jnp.float32,
                    precision=lax.Precision.HIGHEST)
    xs = start * (1.0 / seq)
    ws = tl * (1.0 / seq)

    u = (pw_ref[0:1, :] + pw_ref[1:2, :]).reshape(1, 1, _D)
    v = (pw_ref[2:3, :] + pw_ref[3:4, :]).reshape(1, 1, _D)
    pb = pb_ref[...].reshape(1, 1, _D)
    pos = xs[:, :, None] * u + ws[:, :, None] * v + pb

    a = x2[:, :_D].reshape(hh, seq, _D)
    cc = x2[:, _D:].reshape(hh, seq, _D)
    out_ref[0:hh] = a + pos[0:hh]
    out_ref[hh:bb] = cc + pos[hh:bb]


def _finalize(ids, g2, pos_W, pos_b):
    batch, seq = ids.shape
    bb = 32
    grid = batch // bb
    return pl.pallas_call(
        _finalize_body,
        grid=(grid,),
        in_specs=[
            pl.BlockSpec((bb, seq), lambda i: (i, 0)),
            pl.BlockSpec((bb * seq // 2, 2 * _D), lambda i: (i, 0)),
            pl.BlockSpec((4, _D), lambda i: (0, 0)),
            pl.BlockSpec((1, _D), lambda i: (0, 0)),
        ],
        out_specs=pl.BlockSpec((bb, seq, _D), lambda i: (i, 0, 0)),
        out_shape=jax.ShapeDtypeStruct((batch, seq, _D), jnp.float32),
    )(ids, g2, pos_W, pos_b.reshape(1, _D))


def kernel(input_ids, token_table, pos_W, pos_b):
    b, seq = input_ids.shape
    ids = input_ids.astype(jnp.int32)
    idsf = ids.reshape(b * seq)
    g2 = _make_sc_gather(b * seq)(idsf, token_table)
    return _finalize(ids, g2, pos_W, pos_b)


# speedup vs baseline: 2.7788x; 1.1749x over previous
import functools

import jax
import jax.numpy as jnp
from jax import lax
from jax.experimental import pallas as pl
from jax.experimental.pallas import tpu as pltpu
from jax.experimental.pallas import tpu_sc as plsc

_NC, _NS, _LANES = 2, 16, 16
_NW = _NC * _NS

_D = 64


def _make_sc_gather(n_tokens, chunk=128, nbuf=5):
    npw = n_tokens // _NW
    nchunk = npw // chunk
    ngroup = nchunk // nbuf
    mesh = plsc.VectorSubcoreMesh(core_axis_name="c", subcore_axis_name="s")

    panel = 3200
    scratch = (
        [pltpu.VMEM((chunk,), jnp.int32) for _ in range(nbuf)]
        + [pltpu.VMEM((chunk, _D), jnp.float32) for _ in range(nbuf)]
        + [pltpu.SemaphoreType.DMA for _ in range(2 * nbuf)]
    )

    @functools.partial(
        pl.kernel,
        out_type=jax.ShapeDtypeStruct((n_tokens // 2, 2 * _D), jnp.float32),
        mesh=mesh,
        scratch_types=scratch,
        compiler_params=pltpu.CompilerParams(
            needs_layout_passes=False, use_tc_tiling_on_sc=False
        ),
    )
    def sc_gather(idx_hbm, table_hbm, out_hbm, *scr):
        idx_v = scr[0:nbuf]
        rows_v = scr[nbuf:2 * nbuf]
        gsem = scr[2 * nbuf:3 * nbuf]
        osem = scr[3 * nbuf:4 * nbuf]

        wid = lax.axis_index("s") * _NC + lax.axis_index("c")
        base = wid * npw

        def stage(s, slot):
            pltpu.sync_copy(idx_hbm.at[pl.ds(base + s * chunk, chunk)],
                            idx_v[slot])
            pltpu.async_copy(table_hbm.at[idx_v[slot]], rows_v[slot],
                             gsem[slot])

        def out_copy(s, slot, sem):
            t0 = base + s * chunk
            rem = t0 % (2 * panel)
            odd = rem >= panel
            r0 = (t0 // (2 * panel)) * panel + rem - jnp.where(odd, panel, 0)
            lane0 = jnp.where(odd, _D, 0)
            return pltpu.make_async_copy(
                rows_v[slot],
                out_hbm.at[pl.ds(r0, chunk), pl.ds(lane0, _D)],
                sem)

        ahead = 2
        for b in range(ahead):
            stage(b, b)

        def group_body(g, _):
            for b in range(nbuf):
                s = g * nbuf + b
                slot_n = (b + ahead) % nbuf

                @pl.when(s + ahead < nchunk)
                def _():
                    pltpu.sync_copy(
                        idx_hbm.at[pl.ds(base + (s + ahead) * chunk, chunk)],
                        idx_v[slot_n])

                    @pl.when(s + ahead - nbuf >= 0)
                    def _():
                        out_copy(0, slot_n, osem[slot_n]).wait()

                    pltpu.async_copy(table_hbm.at[idx_v[slot_n]],
                                     rows_v[slot_n], gsem[slot_n])

                pltpu.make_async_copy(
                    table_hbm.at[idx_v[b]], rows_v[b], gsem[b]).wait()
                out_copy(s, b, osem[b]).start()
            return 0

        lax.fori_loop(0, ngroup, group_body, 0)

        for b in range(nbuf):
            out_copy(0, b, osem[b]).wait()

    return sc_gather


def _finalize_body(ids_ref, g2_ref, pw_ref, pb_ref, out_ref):
    bb, seq = ids_ref.shape
    hh = bb // 2
    x2 = g2_ref[...]

    ids = ids_ref[...]
    tl = ((ids % 8) + 1).astype(jnp.float32)
    r = lax.broadcasted_iota(jnp.int32, (seq, seq), 0)
    c = lax.broadcasted_iota(jnp.int32, (seq, seq), 1)
    tri = (r < c).astype(jnp.float32)
    start = jnp.dot(tl, tri, preferred_element_type=jnp.float32,
                    precision=lax.Precision.HIGHEST)
    xs = start * (1.0 / seq)
    ws = tl * (1.0 / seq)

    u = (pw_ref[0:1, :] + pw_ref[1:2, :]).reshape(1, 1, _D)
    v = (pw_ref[2:3, :] + pw_ref[3:4, :]).reshape(1, 1, _D)
    pb = pb_ref[...].reshape(1, 1, _D)
    pos = xs[:, :, None] * u + ws[:, :, None] * v + pb

    pair_rows = 3200
    pieces = []
    for j in range(x2.shape[0] // pair_rows):
        blk = x2[j * pair_rows:(j + 1) * pair_rows]
        pieces.append(blk[:, :_D])
        pieces.append(blk[:, _D:])
    toks = jnp.concatenate(pieces, axis=0)
    res = toks + pos.reshape(bb * seq, _D)
    out_ref[...] = jnp.transpose(res.reshape(bb, seq, _D), (1, 2, 0))


def _finalize(ids, g2, pos_W, pos_b):
    batch, seq = ids.shape
    bb = 128
    grid = batch // bb
    return pl.pallas_call(
        _finalize_body,
        grid=(grid,),
        in_specs=[
            pl.BlockSpec((bb, seq), lambda i: (i, 0)),
            pl.BlockSpec((bb * seq // 2, 2 * _D), lambda i: (i, 0)),
            pl.BlockSpec((4, _D), lambda i: (0, 0)),
            pl.BlockSpec((1, _D), lambda i: (0, 0)),
        ],
        out_specs=pl.BlockSpec((seq, _D, bb), lambda i: (0, 0, i)),
        out_shape=jax.ShapeDtypeStruct((seq, _D, batch), jnp.float32),
    )(ids, g2, pos_W, pos_b.reshape(1, _D))


def kernel(input_ids, token_table, pos_W, pos_b):
    b, seq = input_ids.shape
    ids = input_ids.astype(jnp.int32)
    idsf = ids.reshape(b * seq)
    g2 = _make_sc_gather(b * seq)(idsf, token_table)
    out_t = _finalize(ids, g2, pos_W, pos_b)
    return jnp.transpose(out_t, (2, 0, 1))
